# baseline (speedup 1.0000x reference)
import functools

import jax
import jax.numpy as jnp
from jax import lax
from jax.experimental import pallas as pl
from jax.experimental.pallas import tpu as pltpu
from jax.experimental.pallas import tpu_sc as plsc

B = 1024
N = 20000
N_DN = 1300
N_MBON = 96
NC = 2
NS = 16
NW = NC * NS
BLK = 8
DN_BLKS = (N_DN + BLK - 1) // BLK
DN_TAIL = N_DN - (DN_BLKS - 1) * BLK
MB_BLKS = N_MBON // BLK
TOT_BLKS = DN_BLKS + MB_BLKS
IDX_PAD = TOT_BLKS * BLK
MAX_BLKS_PER_W = (TOT_BLKS + NW - 1) // NW
NBUF = 3


def _sc_body(vt_hbm, dn_idx_hbm, mbon_idx_hbm, dnt_hbm, mbt_hbm,
             cidx_v, g0, g1, g2, sg0, sg1, sg2, so0, so1, so2):
    wid = lax.axis_index("s") * NC + lax.axis_index("c")
    gb = (g0, g1, g2)
    sg = (sg0, sg1, sg2)
    so = (so0, so1, so2)

    pltpu.sync_copy(dn_idx_hbm, cidx_v.at[pl.ds(0, N_DN)])
    tail = cidx_v[pl.ds(N_DN - DN_TAIL, 2 * BLK)]
    cidx_v[pl.ds(N_DN - DN_TAIL, 2 * BLK)] = jnp.where(
        lax.iota(jnp.int32, 2 * BLK) < DN_TAIL, tail, 0)
    pltpu.sync_copy(mbon_idx_hbm, cidx_v.at[pl.ds(DN_BLKS * BLK, N_MBON)])

    def blk_of(k):
        return wid + NW * k

    def issue_gather(k):
        blk = blk_of(k)

        @pl.when(blk < TOT_BLKS)
        def _():
            pltpu.async_copy(vt_hbm.at[cidx_v.at[pl.ds(blk * BLK, BLK)]],
                             gb[k % NBUF], sg[k % NBUF])

    def wait_gather(k):
        blk = blk_of(k)

        @pl.when(blk < TOT_BLKS)
        def _():
            pltpu.make_async_copy(
                vt_hbm.at[cidx_v.at[pl.ds(blk * BLK, BLK)]],
                gb[k % NBUF], sg[k % NBUF]).wait()

    def out_copies(k, blk):
        full_dn = blk < DN_BLKS - 1
        part_dn = blk == DN_BLKS - 1
        is_mb = (blk >= DN_BLKS) & (blk < TOT_BLKS)
        return full_dn, part_dn, is_mb

    def issue_out(k):
        blk = blk_of(k)
        full_dn, part_dn, is_mb = out_copies(k, blk)

        @pl.when(full_dn)
        def _():
            pltpu.async_copy(gb[k % NBUF], dnt_hbm.at[pl.ds(blk * BLK, BLK)],
                             so[k % NBUF])

        @pl.when(part_dn)
        def _():
            pltpu.async_copy(gb[k % NBUF].at[pl.ds(0, DN_TAIL)],
                             dnt_hbm.at[pl.ds((DN_BLKS - 1) * BLK, DN_TAIL)],
                             so[k % NBUF])

        @pl.when(is_mb)
        def _():
            pltpu.async_copy(gb[k % NBUF],
                             mbt_hbm.at[pl.ds((blk - DN_BLKS) * BLK, BLK)],
                             so[k % NBUF])

    def wait_out(k):
        blk = blk_of(k)
        full_dn, part_dn, is_mb = out_copies(k, blk)

        @pl.when(full_dn)
        def _():
            pltpu.make_async_copy(gb[k % NBUF],
                                  dnt_hbm.at[pl.ds(blk * BLK, BLK)],
                                  so[k % NBUF]).wait()

        @pl.when(part_dn)
        def _():
            pltpu.make_async_copy(
                gb[k % NBUF].at[pl.ds(0, DN_TAIL)],
                dnt_hbm.at[pl.ds((DN_BLKS - 1) * BLK, DN_TAIL)],
                so[k % NBUF]).wait()

        @pl.when(is_mb)
        def _():
            pltpu.make_async_copy(gb[k % NBUF],
                                  mbt_hbm.at[pl.ds((blk - DN_BLKS) * BLK, BLK)],
                                  so[k % NBUF]).wait()

    for k in range(min(NBUF, MAX_BLKS_PER_W)):
        issue_gather(k)
    for k in range(MAX_BLKS_PER_W):
        if k >= NBUF:
            wait_out(k - NBUF)
            issue_gather(k)
        wait_gather(k)
        issue_out(k)
    for k in range(max(0, MAX_BLKS_PER_W - NBUF), MAX_BLKS_PER_W):
        wait_out(k)


@jax.jit
def kernel(v, dn_id, mbon_id):
    vt = jnp.swapaxes(v, 0, 1)

    mesh = plsc.VectorSubcoreMesh(core_axis_name="c", subcore_axis_name="s")
    run = pl.kernel(
        _sc_body,
        mesh=mesh,
        compiler_params=pltpu.CompilerParams(needs_layout_passes=False,
                                             use_tc_tiling_on_sc=True,
                                             skip_device_barrier=True,
                                             disable_bounds_checks=True,
                                             disable_semaphore_checks=True),
        out_type=(jax.ShapeDtypeStruct((N_DN, B), jnp.float32),
                  jax.ShapeDtypeStruct((N_MBON, B), jnp.float32)),
        scratch_types=(
            [pltpu.VMEM((IDX_PAD,), jnp.int32)]
            + [pltpu.VMEM((BLK, B), jnp.float32) for _ in range(NBUF)]
            + [pltpu.SemaphoreType.DMA for _ in range(2 * NBUF)]
        ),
    )
    dnt, mbt = run(vt, dn_id.astype(jnp.int32), mbon_id.astype(jnp.int32))
    return jnp.swapaxes(dnt, 0, 1), jnp.swapaxes(mbt, 0, 1)
